# baseline (speedup 1.0000x reference)
import functools

import jax
import jax.numpy as jnp
from jax import lax
from jax.experimental import pallas as pl
from jax.experimental.pallas import tpu as pltpu
from jax.experimental.pallas import tpu_sc as plsc


def _mf_dot(u_id, i_id, user_emb, item_emb):
    B = u_id.shape[0]
    D = user_emb.shape[1]
    info = plsc.get_sparse_core_info()
    NC, NS, L = info.num_cores, info.num_subcores, info.num_lanes
    NW = NC * NS
    n = B // NW

    mesh = plsc.VectorSubcoreMesh(core_axis_name="c", subcore_axis_name="s")

    @functools.partial(
        pl.kernel,
        out_type=jax.ShapeDtypeStruct((B,), jnp.float32),
        mesh=mesh,
        scratch_types=[
            pltpu.VMEM((n,), jnp.int32),
            pltpu.VMEM((n,), jnp.int32),
            pltpu.VMEM((n, D), jnp.float32),
            pltpu.VMEM((n, D), jnp.float32),
            pltpu.VMEM((n,), jnp.float32),
            pltpu.SemaphoreType.DMA,
            pltpu.SemaphoreType.DMA,
        ],
        compiler_params=pltpu.CompilerParams(
            needs_layout_passes=False, use_tc_tiling_on_sc=False),
    )
    def mf_kernel(u_id_hbm, i_id_hbm, u_emb_hbm, i_emb_hbm, out_hbm,
                  u_idx, i_idx, u_rows, i_rows, out_v, sem_u, sem_i):
        wid = lax.axis_index("s") * NC + lax.axis_index("c")
        base = wid * n
        pltpu.sync_copy(u_id_hbm.at[pl.ds(base, n)], u_idx)
        pltpu.sync_copy(i_id_hbm.at[pl.ds(base, n)], i_idx)
        cu = pltpu.async_copy(u_emb_hbm.at[u_idx], u_rows, sem_u)
        ci = pltpu.async_copy(i_emb_hbm.at[i_idx], i_rows, sem_i)
        cu.wait()
        ci.wait()

        lanes = lax.iota(jnp.int32, L)

        def group(g, carry):
            rows = g * L + lanes
            acc = jnp.zeros((L,), jnp.float32)
            for d in range(D):
                col = jnp.full((L,), d, jnp.int32)
                uu = plsc.load_gather(u_rows, [rows, col])
                ii = plsc.load_gather(i_rows, [rows, col])
                acc = acc + uu * ii
            out_v[pl.ds(g * L, L)] = acc
            return carry

        lax.fori_loop(0, n // L, group, 0)
        pltpu.sync_copy(out_v, out_hbm.at[pl.ds(base, n)])

    return mf_kernel(u_id, i_id, user_emb, item_emb)


def kernel(u_id, i_id, user_emb, item_emb, user_bias, item_bias):
    del user_bias, item_bias
    return _mf_dot(u_id.astype(jnp.int32), i_id.astype(jnp.int32),
                   user_emb, item_emb)


# speedup vs baseline: 1.8835x; 1.8835x over previous
import functools

import jax
import jax.numpy as jnp
from jax import lax
from jax.experimental import pallas as pl
from jax.experimental.pallas import tpu as pltpu
from jax.experimental.pallas import tpu_sc as plsc

_L = 16
_W = 128
_RUNS_MAX = 520


def _stage_pair(uk, ik, u_tab, i_tab):
    B = uk.shape[0]
    D, V = u_tab.shape
    info = plsc.get_sparse_core_info()
    NC, NS = info.num_cores, info.num_subcores
    NW = NC * NS
    n = B // NW
    NH = D // _L
    mesh = plsc.VectorSubcoreMesh(core_axis_name="c", subcore_axis_name="s")

    @functools.partial(
        pl.kernel,
        out_type=(jax.ShapeDtypeStruct((B, _W), jnp.float32),
                  jax.ShapeDtypeStruct((B, _W), jnp.float32)),
        mesh=mesh,
        scratch_types=[
            pltpu.VMEM((n,), jnp.int32),
            pltpu.VMEM((n,), jnp.int32),
            pltpu.VMEM((_RUNS_MAX,), jnp.int32),
            pltpu.VMEM((D, _W), jnp.float32),
            pltpu.VMEM((D, _W), jnp.float32),
            pltpu.VMEM((n, _W), jnp.float32),
            pltpu.SemaphoreType.DMA,
            pltpu.SemaphoreType.DMA,
        ],
        compiler_params=pltpu.CompilerParams(
            needs_layout_passes=False, use_tc_tiling_on_sc=True,
            disable_bounds_checks=True),
    )
    def stage_kernel(uk_hbm, ik_hbm, u_tab_hbm, i_tab_hbm, u_gath, i_gath,
                     keys_v, tgs_v, runs_v, blk0, blk1, staged,
                     sem0, sem1):
        wid = lax.axis_index("s") * NC + lax.axis_index("c")
        base = wid * n
        lanes = lax.iota(jnp.int32, _L)

        def splat(x):
            return jnp.full((_L,), x, jnp.int32)

        def sread(ref1d, e):
            v = plsc.load_gather(ref1d, [splat(e)])
            return lax.reduce_max(v, (0,))

        def side(keys_hbm, tab, gath):
            pltpu.sync_copy(keys_hbm.at[pl.ds(base, n)], keys_v)

            for v in range(n // _L):
                k = keys_v[pl.ds(v * _L, _L)]
                tgs_v[pl.ds(v * _L, _L)] = lax.shift_right_logical(k, 7)
            for v in range(_RUNS_MAX // _L):
                runs_v[pl.ds(v * _L, _L)] = splat(n)

            def pa(v, cnt_vec):
                tg = tgs_v[pl.ds(v * _L, _L)]
                pidx = jnp.maximum(v * _L - 1 + lanes, 0)
                prev = plsc.load_gather(tgs_v, [pidx])
                m = tg != prev
                m = jnp.logical_or(m, jnp.logical_and(v == 0, lanes == 0))
                mi = m.astype(jnp.int32)
                excl = plsc.cumsum(mi) - mi
                plsc.store_scatter(runs_v, [cnt_vec + excl], v * _L + lanes,
                                   mask=m)
                return cnt_vec + plsc.all_reduce_population_count(m)
            cnt_vec = lax.fori_loop(0, n // _L, pa, jnp.zeros((_L,), jnp.int32))
            cnt = lax.reduce_max(cnt_vec, (0,))
            cnt2 = (cnt + 1) // 2

            def fetch(j, blk, sem):
                e0 = jnp.minimum(sread(runs_v, j), n - 1)
                tg = sread(tgs_v, e0)
                off = pl.multiple_of(tg * _W, _W)
                return pltpu.async_copy(tab.at[:, pl.ds(off, _W)], blk, sem)

            def wait_for(blk, sem):
                pltpu.make_async_copy(tab.at[:, pl.ds(0, _W)], blk, sem).wait()

            def extract(j, blk):
                e0 = sread(runs_v, j)
                e1 = jnp.minimum(sread(runs_v, j + 1), n)

                def elem(e, c):
                    lane = jnp.bitwise_and(
                        plsc.load_gather(keys_v, [splat(e)]), _W - 1)
                    for h in range(NH):
                        v = plsc.load_gather(blk, [lanes + _L * h, lane])
                        staged[e, pl.ds(_L * h, _L)] = v
                    return c
                lax.fori_loop(e0, e1, elem, 0)

            def kb(j, c):
                fetch(j, blk0, sem0).wait()
                extract(j, blk0)
                return c
            lax.fori_loop(0, cnt, kb, 0)
            pltpu.sync_copy(staged, gath.at[pl.ds(base, n)])

        side(uk_hbm, u_tab_hbm, u_gath)
        side(ik_hbm, i_tab_hbm, i_gath)

    return stage_kernel(uk, ik, u_tab, i_tab)


def _dot(u_gath, i_gath, inv_u, inv_i, D):
    B = u_gath.shape[0]
    info = plsc.get_sparse_core_info()
    NC, NS = info.num_cores, info.num_subcores
    NW = NC * NS
    n = B // NW
    half = n // 2
    mesh = plsc.VectorSubcoreMesh(core_axis_name="c", subcore_axis_name="s")

    @functools.partial(
        pl.kernel,
        out_type=jax.ShapeDtypeStruct((B,), jnp.float32),
        mesh=mesh,
        scratch_types=[
            pltpu.VMEM((half,), jnp.int32),
            pltpu.VMEM((half,), jnp.int32),
            pltpu.VMEM((half, _W), jnp.float32),
            pltpu.VMEM((half, _W), jnp.float32),
            pltpu.VMEM((n,), jnp.float32),
            pltpu.SemaphoreType.DMA,
            pltpu.SemaphoreType.DMA,
        ],
        compiler_params=pltpu.CompilerParams(
            needs_layout_passes=False, use_tc_tiling_on_sc=True,
            disable_bounds_checks=True),
    )
    def dot_kernel(u_gath_hbm, i_gath_hbm, inv_u_hbm, inv_i_hbm, out_hbm,
                   iu_v, ii_v, u_buf, i_buf, out_v, semu, semi):
        wid = lax.axis_index("s") * NC + lax.axis_index("c")
        base = wid * n
        lanes = lax.iota(jnp.int32, _L)

        for hb in range(2):
            off = base + hb * half
            pltpu.sync_copy(inv_u_hbm.at[pl.ds(off, half)], iu_v)
            pltpu.sync_copy(inv_i_hbm.at[pl.ds(off, half)], ii_v)
            cu = pltpu.async_copy(u_gath_hbm.at[iu_v], u_buf, semu)
            ci = pltpu.async_copy(i_gath_hbm.at[ii_v], i_buf, semi)
            cu.wait()
            ci.wait()

            def group(g, c):
                rows = g * _L + lanes
                acc = jnp.zeros((_L,), jnp.float32)
                for d in range(D):
                    col = jnp.full((_L,), d, jnp.int32)
                    uu = plsc.load_gather(u_buf, [rows, col])
                    ii = plsc.load_gather(i_buf, [rows, col])
                    acc = acc + uu * ii
                out_v[pl.ds(hb * half + g * _L, _L)] = acc
                return c
            lax.fori_loop(0, half // _L, group, 0)
        pltpu.sync_copy(out_v, out_hbm.at[pl.ds(base, n)])

    return dot_kernel(u_gath, i_gath, inv_u, inv_i)


def kernel(u_id, i_id, user_emb, item_emb, user_bias, item_bias):
    del user_bias, item_bias
    u_id = u_id.astype(jnp.int32)
    i_id = i_id.astype(jnp.int32)
    B = u_id.shape[0]
    D = user_emb.shape[1]
    pos = lax.iota(jnp.int32, B)
    uk, up = lax.sort((u_id, pos), num_keys=1)
    ik, ip = lax.sort((i_id, pos), num_keys=1)
    inv_u = lax.sort((up, pos), num_keys=1)[1]
    inv_i = lax.sort((ip, pos), num_keys=1)[1]
    u_gath, i_gath = _stage_pair(uk, ik, user_emb.T, item_emb.T)
    return _dot(u_gath, i_gath, inv_u, inv_i, D)


# speedup vs baseline: 2.9678x; 1.5757x over previous
import functools

import jax
import jax.numpy as jnp
from jax import lax
from jax.experimental import pallas as pl
from jax.experimental.pallas import tpu as pltpu
from jax.experimental.pallas import tpu_sc as plsc

_L = 16
_W = 128
_RUNS_MAX = 520


def _stage_pair(uk, ik, u_tab, i_tab):
    B = uk.shape[0]
    D, V = u_tab.shape
    info = plsc.get_sparse_core_info()
    NC, NS = info.num_cores, info.num_subcores
    NW = NC * NS
    n = B // NW
    NH = D // _L
    mesh = plsc.VectorSubcoreMesh(core_axis_name="c", subcore_axis_name="s")

    @functools.partial(
        pl.kernel,
        out_type=(jax.ShapeDtypeStruct((B, _W), jnp.float32),
                  jax.ShapeDtypeStruct((B, _W), jnp.float32)),
        mesh=mesh,
        scratch_types=[
            pltpu.VMEM((n,), jnp.int32),
            pltpu.VMEM((n,), jnp.int32),
            pltpu.VMEM((_RUNS_MAX,), jnp.int32),
            pltpu.VMEM((D, _W), jnp.float32),
            pltpu.VMEM((D, _W), jnp.float32),
            pltpu.VMEM((D, _W), jnp.float32),
            pltpu.VMEM((D, _W), jnp.float32),
            pltpu.VMEM((n, _W), jnp.float32),
            pltpu.SemaphoreType.DMA,
        ],
        compiler_params=pltpu.CompilerParams(
            needs_layout_passes=False, use_tc_tiling_on_sc=True,
            disable_bounds_checks=True),
    )
    def stage_kernel(uk_hbm, ik_hbm, u_tab_hbm, i_tab_hbm, u_gath, i_gath,
                     keys_v, tgs_v, runs_v, blk0, blk1, blk2, blk3, staged,
                     sem0):
        blks = (blk0, blk1, blk2, blk3)
        wid = lax.axis_index("s") * NC + lax.axis_index("c")
        base = wid * n
        lanes = lax.iota(jnp.int32, _L)

        def splat(x):
            return jnp.full((_L,), x, jnp.int32)

        def sread(ref1d, e):
            v = plsc.load_gather(ref1d, [splat(e)])
            return lax.reduce_max(v, (0,))

        def side(keys_hbm, tab, gath):
            pltpu.sync_copy(keys_hbm.at[pl.ds(base, n)], keys_v)

            for v in range(n // _L):
                k = keys_v[pl.ds(v * _L, _L)]
                tgs_v[pl.ds(v * _L, _L)] = lax.shift_right_logical(k, 7)
            for v in range(_RUNS_MAX // _L):
                runs_v[pl.ds(v * _L, _L)] = splat(n)

            def pa(v, cnt_vec):
                tg = tgs_v[pl.ds(v * _L, _L)]
                pidx = jnp.maximum(v * _L - 1 + lanes, 0)
                prev = plsc.load_gather(tgs_v, [pidx])
                m = tg != prev
                m = jnp.logical_or(m, jnp.logical_and(v == 0, lanes == 0))
                mi = m.astype(jnp.int32)
                excl = plsc.cumsum(mi) - mi
                plsc.store_scatter(runs_v, [cnt_vec + excl], v * _L + lanes,
                                   mask=m)
                return cnt_vec + plsc.all_reduce_population_count(m)
            cnt_vec = lax.fori_loop(0, n // _L, pa, jnp.zeros((_L,), jnp.int32))
            cnt = lax.reduce_max(cnt_vec, (0,))
            cnt2 = (cnt + 1) // 2

            def fetch(j, blk, sem):
                e0 = jnp.minimum(sread(runs_v, j), n - 1)
                tg = sread(tgs_v, e0)
                off = pl.multiple_of(tg * _W, _W)
                return pltpu.async_copy(tab.at[:, pl.ds(off, _W)], blk, sem)

            def wait_for(blk, sem):
                pltpu.make_async_copy(tab.at[:, pl.ds(0, _W)], blk, sem).wait()

            def extract(j, blk):
                e0 = sread(runs_v, j)
                e1 = jnp.minimum(sread(runs_v, j + 1), n)

                def elem(e, c):
                    lane = jnp.bitwise_and(
                        plsc.load_gather(keys_v, [splat(e)]), _W - 1)
                    for h in range(NH):
                        v = plsc.load_gather(blk, [lanes + _L * h, lane])
                        staged[e, pl.ds(_L * h, _L)] = v
                    return c
                lax.fori_loop(e0, e1, elem, 0)

            def kb(k, c):
                j = 4 * k
                hs = [fetch(j + t, blks[t], sem0) for t in range(4)]
                for t in range(4):
                    hs[t].wait()
                for t in range(4):
                    extract(j + t, blks[t])
                return c
            lax.fori_loop(0, (cnt + 3) // 4, kb, 0)
            pltpu.sync_copy(staged, gath.at[pl.ds(base, n)])

        side(uk_hbm, u_tab_hbm, u_gath)
        side(ik_hbm, i_tab_hbm, i_gath)

    return stage_kernel(uk, ik, u_tab, i_tab)


def _dot(u_gath, i_gath, inv_u, inv_i, D):
    B = u_gath.shape[0]
    info = plsc.get_sparse_core_info()
    NC, NS = info.num_cores, info.num_subcores
    NW = NC * NS
    n = B // NW
    half = n // 2
    mesh = plsc.VectorSubcoreMesh(core_axis_name="c", subcore_axis_name="s")

    @functools.partial(
        pl.kernel,
        out_type=jax.ShapeDtypeStruct((B,), jnp.float32),
        mesh=mesh,
        scratch_types=[
            pltpu.VMEM((half,), jnp.int32),
            pltpu.VMEM((half,), jnp.int32),
            pltpu.VMEM((half, _W), jnp.float32),
            pltpu.VMEM((half, _W), jnp.float32),
            pltpu.VMEM((n,), jnp.float32),
            pltpu.SemaphoreType.DMA,
            pltpu.SemaphoreType.DMA,
        ],
        compiler_params=pltpu.CompilerParams(
            needs_layout_passes=False, use_tc_tiling_on_sc=True,
            disable_bounds_checks=True),
    )
    def dot_kernel(u_gath_hbm, i_gath_hbm, inv_u_hbm, inv_i_hbm, out_hbm,
                   iu_v, ii_v, u_buf, i_buf, out_v, semu, semi):
        wid = lax.axis_index("s") * NC + lax.axis_index("c")
        base = wid * n
        lanes = lax.iota(jnp.int32, _L)

        for hb in range(2):
            off = base + hb * half
            pltpu.sync_copy(inv_u_hbm.at[pl.ds(off, half)], iu_v)
            pltpu.sync_copy(inv_i_hbm.at[pl.ds(off, half)], ii_v)
            cu = pltpu.async_copy(u_gath_hbm.at[iu_v], u_buf, semu)
            ci = pltpu.async_copy(i_gath_hbm.at[ii_v], i_buf, semi)
            cu.wait()
            ci.wait()

            def group(g, c):
                rows = g * _L + lanes
                acc = jnp.zeros((_L,), jnp.float32)
                for d in range(D):
                    col = jnp.full((_L,), d, jnp.int32)
                    uu = plsc.load_gather(u_buf, [rows, col])
                    ii = plsc.load_gather(i_buf, [rows, col])
                    acc = acc + uu * ii
                out_v[pl.ds(hb * half + g * _L, _L)] = acc
                return c
            lax.fori_loop(0, half // _L, group, 0)
        pltpu.sync_copy(out_v, out_hbm.at[pl.ds(base, n)])

    return dot_kernel(u_gath, i_gath, inv_u, inv_i)


def kernel(u_id, i_id, user_emb, item_emb, user_bias, item_bias):
    del user_bias, item_bias
    u_id = u_id.astype(jnp.int32)
    i_id = i_id.astype(jnp.int32)
    B = u_id.shape[0]
    D = user_emb.shape[1]
    pos = lax.iota(jnp.int32, B)
    uk, up = lax.sort((u_id, pos), num_keys=1)
    ik, ip = lax.sort((i_id, pos), num_keys=1)
    inv_u = lax.sort((up, pos), num_keys=1)[1]
    inv_i = lax.sort((ip, pos), num_keys=1)[1]
    u_gath, i_gath = _stage_pair(uk, ik, user_emb.T, item_emb.T)
    return _dot(u_gath, i_gath, inv_u, inv_i, D)


# speedup vs baseline: 3.0810x; 1.0381x over previous
import functools

import jax
import jax.numpy as jnp
from jax import lax
from jax.experimental import pallas as pl
from jax.experimental.pallas import tpu as pltpu
from jax.experimental.pallas import tpu_sc as plsc

_L = 16
_W = 128
_RUNS_MAX = 544


def _stage_pair(uk, ik, u_tab, i_tab):
    B = uk.shape[0]
    D, V = u_tab.shape
    info = plsc.get_sparse_core_info()
    NC, NS = info.num_cores, info.num_subcores
    NW = NC * NS
    n = B // NW
    NH = D // _L
    mesh = plsc.VectorSubcoreMesh(core_axis_name="c", subcore_axis_name="s")

    @functools.partial(
        pl.kernel,
        out_type=(jax.ShapeDtypeStruct((B, _W), jnp.float32),
                  jax.ShapeDtypeStruct((B, _W), jnp.float32)),
        mesh=mesh,
        scratch_types=[
            pltpu.VMEM((n,), jnp.int32),
            pltpu.VMEM((n,), jnp.int32),
            pltpu.VMEM((_RUNS_MAX,), jnp.int32),
            pltpu.VMEM((D, _W), jnp.float32),
            pltpu.VMEM((D, _W), jnp.float32),
            pltpu.VMEM((D, _W), jnp.float32),
            pltpu.VMEM((D, _W), jnp.float32),
            pltpu.VMEM((D, _W), jnp.float32),
            pltpu.VMEM((D, _W), jnp.float32),
            pltpu.VMEM((n, _W), jnp.float32),
            pltpu.SemaphoreType.DMA,
            pltpu.SemaphoreType.DMA,
        ],
        compiler_params=pltpu.CompilerParams(
            needs_layout_passes=False, use_tc_tiling_on_sc=True,
            disable_bounds_checks=True),
    )
    def stage_kernel(uk_hbm, ik_hbm, u_tab_hbm, i_tab_hbm, u_gath, i_gath,
                     keys_v, tgs_v, runs_v, ba0, ba1, ba2, bb0, bb1, bb2,
                     staged, semA, semB):
        bufA = (ba0, ba1, ba2)
        bufB = (bb0, bb1, bb2)
        wid = lax.axis_index("s") * NC + lax.axis_index("c")
        base = wid * n
        lanes = lax.iota(jnp.int32, _L)

        def splat(x):
            return jnp.full((_L,), x, jnp.int32)

        def sread(ref1d, e):
            v = plsc.load_gather(ref1d, [splat(e)])
            return lax.reduce_max(v, (0,))

        def side(keys_hbm, tab, gath):
            pltpu.sync_copy(keys_hbm.at[pl.ds(base, n)], keys_v)

            for v in range(n // _L):
                k = keys_v[pl.ds(v * _L, _L)]
                tgs_v[pl.ds(v * _L, _L)] = lax.shift_right_logical(k, 7)
            for v in range(_RUNS_MAX // _L):
                runs_v[pl.ds(v * _L, _L)] = splat(n)

            def pa(v, cnt_vec):
                tg = tgs_v[pl.ds(v * _L, _L)]
                pidx = jnp.maximum(v * _L - 1 + lanes, 0)
                prev = plsc.load_gather(tgs_v, [pidx])
                m = tg != prev
                m = jnp.logical_or(m, jnp.logical_and(v == 0, lanes == 0))
                mi = m.astype(jnp.int32)
                excl = plsc.cumsum(mi) - mi
                plsc.store_scatter(runs_v, [cnt_vec + excl], v * _L + lanes,
                                   mask=m)
                return cnt_vec + plsc.all_reduce_population_count(m)
            cnt_vec = lax.fori_loop(0, n // _L, pa, jnp.zeros((_L,), jnp.int32))
            cnt = lax.reduce_max(cnt_vec, (0,))
            cnt2 = (cnt + 1) // 2

            def fetch(j, blk, sem):
                e0 = jnp.minimum(sread(runs_v, j), n - 1)
                tg = sread(tgs_v, e0)
                off = pl.multiple_of(tg * _W, _W)
                return pltpu.async_copy(tab.at[:, pl.ds(off, _W)], blk, sem)

            def wait_for(blk, sem):
                pltpu.make_async_copy(tab.at[:, pl.ds(0, _W)], blk, sem).wait()

            def extract(j, blk):
                e0 = sread(runs_v, j)
                e1 = jnp.minimum(sread(runs_v, j + 1), n)

                def elem(e, c):
                    lane = jnp.bitwise_and(
                        plsc.load_gather(keys_v, [splat(e)]), _W - 1)
                    for h in range(NH):
                        v = plsc.load_gather(blk, [lanes + _L * h, lane])
                        staged[e, pl.ds(_L * h, _L)] = v
                    return c
                lax.fori_loop(e0, e1, elem, 0)

            NG = 6
            bufs = bufA + bufB

            def kb(k, c):
                j = NG * k
                hs = [fetch(j + t, bufs[t], semA) for t in range(NG)]
                for t in range(NG):
                    hs[t].wait()
                for t in range(NG):
                    extract(j + t, bufs[t])
                return c
            lax.fori_loop(0, (cnt + NG - 1) // NG, kb, 0)
            pltpu.sync_copy(staged, gath.at[pl.ds(base, n)])

        side(uk_hbm, u_tab_hbm, u_gath)
        side(ik_hbm, i_tab_hbm, i_gath)

    return stage_kernel(uk, ik, u_tab, i_tab)


def _dot(u_gath, i_gath, inv_u, inv_i, D):
    B = u_gath.shape[0]
    info = plsc.get_sparse_core_info()
    NC, NS = info.num_cores, info.num_subcores
    NW = NC * NS
    n = B // NW
    half = n // 2
    mesh = plsc.VectorSubcoreMesh(core_axis_name="c", subcore_axis_name="s")

    @functools.partial(
        pl.kernel,
        out_type=jax.ShapeDtypeStruct((B,), jnp.float32),
        mesh=mesh,
        scratch_types=[
            pltpu.VMEM((half,), jnp.int32),
            pltpu.VMEM((half,), jnp.int32),
            pltpu.VMEM((half, _W), jnp.float32),
            pltpu.VMEM((half, _W), jnp.float32),
            pltpu.VMEM((n,), jnp.float32),
            pltpu.SemaphoreType.DMA,
            pltpu.SemaphoreType.DMA,
        ],
        compiler_params=pltpu.CompilerParams(
            needs_layout_passes=False, use_tc_tiling_on_sc=True,
            disable_bounds_checks=True),
    )
    def dot_kernel(u_gath_hbm, i_gath_hbm, inv_u_hbm, inv_i_hbm, out_hbm,
                   iu_v, ii_v, u_buf, i_buf, out_v, semu, semi):
        wid = lax.axis_index("s") * NC + lax.axis_index("c")
        base = wid * n
        lanes = lax.iota(jnp.int32, _L)

        for hb in range(2):
            off = base + hb * half
            pltpu.sync_copy(inv_u_hbm.at[pl.ds(off, half)], iu_v)
            pltpu.sync_copy(inv_i_hbm.at[pl.ds(off, half)], ii_v)
            cu = pltpu.async_copy(u_gath_hbm.at[iu_v], u_buf, semu)
            ci = pltpu.async_copy(i_gath_hbm.at[ii_v], i_buf, semi)
            cu.wait()
            ci.wait()

            def group(g, c):
                rows = g * _L + lanes
                acc = jnp.zeros((_L,), jnp.float32)
                for d in range(D):
                    col = jnp.full((_L,), d, jnp.int32)
                    uu = plsc.load_gather(u_buf, [rows, col])
                    ii = plsc.load_gather(i_buf, [rows, col])
                    acc = acc + uu * ii
                out_v[pl.ds(hb * half + g * _L, _L)] = acc
                return c
            lax.fori_loop(0, half // _L, group, 0)
        pltpu.sync_copy(out_v, out_hbm.at[pl.ds(base, n)])

    return dot_kernel(u_gath, i_gath, inv_u, inv_i)


def kernel(u_id, i_id, user_emb, item_emb, user_bias, item_bias):
    del user_bias, item_bias
    u_id = u_id.astype(jnp.int32)
    i_id = i_id.astype(jnp.int32)
    B = u_id.shape[0]
    D = user_emb.shape[1]
    pos = lax.iota(jnp.int32, B)
    uk, up = lax.sort((u_id, pos), num_keys=1)
    ik, ip = lax.sort((i_id, pos), num_keys=1)
    inv_u = lax.sort((up, pos), num_keys=1)[1]
    inv_i = lax.sort((ip, pos), num_keys=1)[1]
    u_gath, i_gath = _stage_pair(uk, ik, user_emb.T, item_emb.T)
    return _dot(u_gath, i_gath, inv_u, inv_i, D)
